# Initial kernel scaffold; baseline (speedup 1.0000x reference)
import jax
import jax.numpy as jnp
from jax.experimental import pallas as pl


def kernel(x, edge_index, conv1_weight, conv1_bias, fc1_weight, fc1_bias, fc2_weight, fc2_bias, fc3_weight, fc3_bias, fc4_weight, fc4_bias):
    raise NotImplementedError("write your pallas kernel here")


# speedup vs baseline: 13.1104x; 13.1104x over previous
import functools

import jax
import jax.numpy as jnp
from jax import lax
from jax.experimental import pallas as pl
from jax.experimental.pallas import tpu as pltpu
from jax.experimental.pallas import tpu_sc as plsc

N = 10000
E = 320000
H = 128
NC, NS = 2, 16
NW = NC * NS

EPT = E // NW
EPS = E // NS
CHUNK = 80
RPT = 632

_MESH = plsc.VectorSubcoreMesh(
    core_axis_name="c", subcore_axis_name="s", num_cores=NC, num_subcores=NS)


@functools.partial(
    pl.kernel,
    out_type=jax.ShapeDtypeStruct((NW, N), jnp.float32),
    mesh=_MESH,
    compiler_params=pltpu.CompilerParams(needs_layout_passes=False),
    scratch_types=[
        pltpu.VMEM((EPT,), jnp.int32),
        pltpu.VMEM((N,), jnp.float32),
    ],
)
def _deg_kernel(dst_hbm, out_hbm, idx_v, acc_v):
    c = lax.axis_index("c")
    s = lax.axis_index("s")
    wid = s * NC + c

    zeros = jnp.zeros((16,), jnp.float32)

    def zbody(i, carry):
        acc_v[pl.ds(i * 16, 16)] = zeros
        return carry

    lax.fori_loop(0, N // 16, zbody, 0)

    pltpu.sync_copy(dst_hbm.at[pl.ds(wid * EPT, EPT)], idx_v)

    ones = jnp.ones((16,), jnp.float32)

    def body(i, carry):
        idx = idx_v[pl.ds(i * 16, 16)]
        plsc.addupdate_scatter(acc_v, [idx], ones)
        return carry

    lax.fori_loop(0, EPT // 16, body, 0)

    pltpu.sync_copy(acc_v, out_hbm.at[wid])


def _h2_body(x_ref, w_ref, degpt_ref, h2_ref, dinv_ref):
    deg = jnp.sum(degpt_ref[...], axis=1, keepdims=True) + 1.0
    dinv = lax.rsqrt(deg)
    h = jnp.dot(x_ref[...], w_ref[...], preferred_element_type=jnp.float32)
    h2_ref[...] = (h * dinv)[None]
    dinv_ref[...] = dinv


def _h2_call(x, w, degpt):
    return pl.pallas_call(
        _h2_body,
        grid=(10, 2),
        in_specs=[
            pl.BlockSpec((1000, 128), lambda i, c: (i, 0)),
            pl.BlockSpec((128, H), lambda i, c: (0, c)),
            pl.BlockSpec((1000, NW), lambda i, c: (i, 0)),
        ],
        out_specs=[
            pl.BlockSpec((1, 1000, H), lambda i, c: (c, i, 0)),
            pl.BlockSpec((1000, 1), lambda i, c: (i, 0)),
        ],
        out_shape=[
            jax.ShapeDtypeStruct((NC, N, H), jnp.float32),
            jax.ShapeDtypeStruct((N, 1), jnp.float32),
        ],
    )(x, w, degpt)


@functools.partial(
    pl.kernel,
    out_type=jax.ShapeDtypeStruct((NC * N, H), jnp.float32),
    mesh=_MESH,
    scratch_types=[
        pltpu.VMEM_SHARED((N, H), jnp.float32),
        pltpu.VMEM((CHUNK,), jnp.int32),
        pltpu.VMEM((CHUNK,), jnp.int32),
        pltpu.VMEM((CHUNK, H), jnp.float32),
        pltpu.SemaphoreType.DMA,
    ],
)
def _scat_kernel(h2_hbm, src_hbm, dst_hbm, out_hbm, acc_s, sidx, didx, rows, sem):
    c = lax.axis_index("c")
    s = lax.axis_index("s")
    off = jnp.minimum(s * RPT, N - RPT)
    coff = c * N

    pltpu.sync_copy(h2_hbm.at[pl.ds(coff + off, RPT), :], acc_s.at[pl.ds(off, RPT), :])
    plsc.subcore_barrier()

    e0 = s * EPS

    def body(j, carry):
        base = e0 + j * CHUNK
        pltpu.sync_copy(src_hbm.at[pl.ds(base, CHUNK)], sidx)
        pltpu.sync_copy(dst_hbm.at[pl.ds(base, CHUNK)], didx)
        for t in range(CHUNK // 16):
            sl = pl.ds(t * 16, 16)
            sidx[sl] = sidx[sl] + coff
        pltpu.async_copy(h2_hbm.at[sidx], rows, sem).wait()
        pltpu.sync_copy(rows, acc_s.at[didx], add=True)
        return carry

    lax.fori_loop(0, EPS // CHUNK, body, 0)
    plsc.subcore_barrier()

    pltpu.sync_copy(acc_s.at[pl.ds(off, RPT), :], out_hbm.at[pl.ds(coff + off, RPT), :])


def _fin_body(agg_ref, dinv_ref, bias_ref, w1_ref, b1_ref, w2_ref, b2_ref,
              w3_ref, b3_ref, w4_ref, b4_ref,
              g1_ref, be1_ref, g2_ref, be2_ref, s_acc):
    i = pl.program_id(0)

    @pl.when(i == 0)
    def _():
        s_acc[...] = jnp.zeros_like(s_acc)

    t = agg_ref[...] * dinv_ref[...][None] + bias_ref[...]
    t = jnp.maximum(t, 0.0)
    s_acc[...] += jnp.sum(t, axis=1)

    @pl.when(i == pl.num_programs(0) - 1)
    def _():
        s0 = s_acc[0:1, :]
        s1 = s_acc[1:2, :]

        def head(w_ref, b_ref):
            w = w_ref[...]
            dn = (((1,), (1,)), ((), ()))
            r = (lax.dot_general(s0, w[:, :H], dn, preferred_element_type=jnp.float32)
                 + lax.dot_general(s1, w[:, H:], dn, preferred_element_type=jnp.float32))
            return jnp.tanh(r + b_ref[...])

        g1_ref[...] = head(w1_ref, b1_ref)
        be1_ref[...] = head(w2_ref, b2_ref)
        g2_ref[...] = head(w3_ref, b3_ref)
        be2_ref[...] = head(w4_ref, b4_ref)


def _fin_call(agg, dinv, bias2, w1, b1, w2, b2, w3, b3, w4, b4):
    full = lambda shape: pl.BlockSpec(shape, lambda i: tuple(0 for _ in shape))
    return pl.pallas_call(
        _fin_body,
        grid=(10,),
        in_specs=[
            pl.BlockSpec((NC, 1000, H), lambda i: (0, i, 0)),
            pl.BlockSpec((1000, 1), lambda i: (i, 0)),
            full((NC, 1, H)),
            full((256, 256)), full((1, 256)),
            full((256, 256)), full((1, 256)),
            full((128, 256)), full((1, 128)),
            full((128, 256)), full((1, 128)),
        ],
        out_specs=[full((1, 256)), full((1, 256)), full((1, 128)), full((1, 128))],
        out_shape=[
            jax.ShapeDtypeStruct((1, 256), jnp.float32),
            jax.ShapeDtypeStruct((1, 256), jnp.float32),
            jax.ShapeDtypeStruct((1, 128), jnp.float32),
            jax.ShapeDtypeStruct((1, 128), jnp.float32),
        ],
        scratch_shapes=[pltpu.VMEM((NC, H), jnp.float32)],
    )(agg, dinv, bias2, w1, b1, w2, b2, w3, b3, w4, b4)


def kernel(x, edge_index, conv1_weight, conv1_bias, fc1_weight, fc1_bias,
           fc2_weight, fc2_bias, fc3_weight, fc3_bias, fc4_weight, fc4_bias):
    src = edge_index[0]
    dst = edge_index[1]

    degp = _deg_kernel(dst)
    degpt = degp.T
    h2, dinv = _h2_call(x, conv1_weight, degpt)
    aggflat = _scat_kernel(h2.reshape(NC * N, H), src, dst)
    agg = aggflat.reshape(NC, N, H)

    bias2 = conv1_bias.reshape(NC, 1, H)
    g1, b1, g2, b2 = _fin_call(
        agg, dinv, bias2,
        fc1_weight, fc1_bias.reshape(1, -1),
        fc2_weight, fc2_bias.reshape(1, -1),
        fc3_weight, fc3_bias.reshape(1, -1),
        fc4_weight, fc4_bias.reshape(1, -1),
    )
    return (g1.reshape(-1), b1.reshape(-1), g2.reshape(-1), b2.reshape(-1))


# speedup vs baseline: 24.2376x; 1.8487x over previous
import functools

import jax
import jax.numpy as jnp
from jax import lax
from jax.experimental import pallas as pl
from jax.experimental.pallas import tpu as pltpu
from jax.experimental.pallas import tpu_sc as plsc

N = 10000
E = 320000
H = 128
NC, NS = 2, 16
NW = NC * NS

EPT = E // NW
EPS = E // NS
CHUNK = 80
NCH = EPS // CHUNK
RING = 2
NGRP = NCH // RING
RPT = 632

_MESH = plsc.VectorSubcoreMesh(
    core_axis_name="c", subcore_axis_name="s", num_cores=NC, num_subcores=NS)


@functools.partial(
    pl.kernel,
    out_type=jax.ShapeDtypeStruct((NW, N), jnp.float32),
    mesh=_MESH,
    compiler_params=pltpu.CompilerParams(needs_layout_passes=False),
    scratch_types=[
        pltpu.VMEM((EPT,), jnp.int32),
        pltpu.VMEM((N,), jnp.float32),
    ],
)
def _deg_kernel(dst_hbm, out_hbm, idx_v, acc_v):
    c = lax.axis_index("c")
    s = lax.axis_index("s")
    wid = s * NC + c

    zeros = jnp.zeros((16,), jnp.float32)

    def zbody(i, carry):
        acc_v[pl.ds(i * 16, 16)] = zeros
        return carry

    lax.fori_loop(0, N // 16, zbody, 0)

    pltpu.sync_copy(dst_hbm.at[pl.ds(wid * EPT, EPT)], idx_v)

    ones = jnp.ones((16,), jnp.float32)

    def body(i, carry):
        idx = idx_v[pl.ds(i * 16, 16)]
        plsc.addupdate_scatter(acc_v, [idx], ones)
        return carry

    lax.fori_loop(0, EPT // 16, body, 0)

    pltpu.sync_copy(acc_v, out_hbm.at[wid])


def _h2_body(x_ref, w_ref, degpt_ref, h2_ref, dinv_ref):
    deg = jnp.sum(degpt_ref[...], axis=1, keepdims=True) + 1.0
    dinv = lax.rsqrt(deg)
    h = jnp.dot(x_ref[...], w_ref[...], preferred_element_type=jnp.float32)
    h2_ref[...] = (h * dinv)[None]
    dinv_ref[...] = dinv


def _h2_call(x, w, degpt):
    return pl.pallas_call(
        _h2_body,
        grid=(10, 2),
        in_specs=[
            pl.BlockSpec((1000, 128), lambda i, c: (i, 0)),
            pl.BlockSpec((128, H), lambda i, c: (0, c)),
            pl.BlockSpec((1000, NW), lambda i, c: (i, 0)),
        ],
        out_specs=[
            pl.BlockSpec((1, 1000, H), lambda i, c: (c, i, 0)),
            pl.BlockSpec((1000, 1), lambda i, c: (i, 0)),
        ],
        out_shape=[
            jax.ShapeDtypeStruct((NC, N, H), jnp.float32),
            jax.ShapeDtypeStruct((N, 1), jnp.float32),
        ],
    )(x, w, degpt)


@functools.partial(
    pl.kernel,
    out_type=jax.ShapeDtypeStruct((NC, N, H), jnp.float32),
    mesh=_MESH,
    scratch_types=(
        [pltpu.VMEM_SHARED((N, H), jnp.float32)]
        + [pltpu.VMEM((CHUNK, H), jnp.float32) for _ in range(RING)]
        + [pltpu.VMEM((CHUNK,), jnp.int32) for _ in range(3 * RING)]
        + [pltpu.SemaphoreType.DMA for _ in range(3 * RING)]
    ),
)
def _scat_kernel(h2_hbm, combr_hbm, out_hbm, acc_s, *rest):
    rows = rest[:RING]
    cidx = rest[RING:2 * RING]
    sidx = rest[2 * RING:3 * RING]
    didx = rest[3 * RING:4 * RING]
    gsem = rest[4 * RING:5 * RING]
    ssem = rest[5 * RING:6 * RING]
    csem = rest[6 * RING:]
    c = lax.axis_index("c")
    s = lax.axis_index("s")
    off = jnp.minimum(s * RPT, N - RPT)

    pltpu.sync_copy(h2_hbm.at[c, pl.ds(off, RPT), :], acc_s.at[pl.ds(off, RPT), :])
    plsc.subcore_barrier()

    def unpack(b):
        for t in range(CHUNK // 16):
            sl = pl.ds(t * 16, 16)
            v = cidx[b][sl]
            sidx[b][sl] = jnp.bitwise_and(v, 16383)
            didx[b][sl] = lax.shift_right_logical(v, 14)

    def run(tbl):
        for b in range(RING):
            pltpu.async_copy(combr_hbm.at[s, b], cidx[b], csem[b])
        for b in range(RING):
            pltpu.make_async_copy(combr_hbm.at[s, 0], cidx[b], csem[b]).wait()
            unpack(b)
            pltpu.async_copy(tbl.at[sidx[b]], rows[b], gsem[b])
            pltpu.async_copy(combr_hbm.at[s, RING + b], cidx[b], csem[b])

        def body(k, carry):
            for b in range(RING):
                pltpu.make_async_copy(tbl.at[pl.ds(0, CHUNK)], rows[b], gsem[b]).wait()
                pltpu.async_copy(rows[b], acc_s.at[didx[b]], ssem[b], add=True)

            @pl.when(k < NGRP - 1)
            def _():
                for b in range(RING):
                    j2 = (k + 1) * RING + b
                    pltpu.make_async_copy(rows[b], acc_s.at[didx[b]], ssem[b]).wait()
                    pltpu.make_async_copy(combr_hbm.at[s, 0], cidx[b], csem[b]).wait()
                    unpack(b)
                    pltpu.async_copy(tbl.at[sidx[b]], rows[b], gsem[b])

                    @pl.when(k < NGRP - 2)
                    def _():
                        pltpu.async_copy(
                            combr_hbm.at[s, j2 + RING], cidx[b], csem[b])

            return carry

        lax.fori_loop(0, NGRP, body, 0)
        for b in range(RING):
            pltpu.make_async_copy(rows[b], acc_s.at[didx[b]], ssem[b]).wait()

    @pl.when(c == 0)
    def _():
        run(h2_hbm.at[0])

    @pl.when(c == 1)
    def _():
        run(h2_hbm.at[1])

    plsc.subcore_barrier()
    pltpu.sync_copy(acc_s.at[pl.ds(off, RPT), :], out_hbm.at[c, pl.ds(off, RPT), :])


def _fin_body(agg_ref, dinv_ref, bias_ref, w1_ref, b1_ref, w2_ref, b2_ref,
              w3_ref, b3_ref, w4_ref, b4_ref,
              g1_ref, be1_ref, g2_ref, be2_ref, s_acc):
    i = pl.program_id(0)

    @pl.when(i == 0)
    def _():
        s_acc[...] = jnp.zeros_like(s_acc)

    t = agg_ref[...] * dinv_ref[...][None] + bias_ref[...]
    t = jnp.maximum(t, 0.0)
    s_acc[...] += jnp.sum(t, axis=1)

    @pl.when(i == pl.num_programs(0) - 1)
    def _():
        s0 = s_acc[0:1, :]
        s1 = s_acc[1:2, :]

        def head(w_ref, b_ref):
            w = w_ref[...]
            dn = (((1,), (1,)), ((), ()))
            r = (lax.dot_general(s0, w[:, :H], dn, preferred_element_type=jnp.float32)
                 + lax.dot_general(s1, w[:, H:], dn, preferred_element_type=jnp.float32))
            return jnp.tanh(r + b_ref[...])

        g1_ref[...] = head(w1_ref, b1_ref)
        be1_ref[...] = head(w2_ref, b2_ref)
        g2_ref[...] = head(w3_ref, b3_ref)
        be2_ref[...] = head(w4_ref, b4_ref)


def _fin_call(agg, dinv, bias2, w1, b1, w2, b2, w3, b3, w4, b4):
    full = lambda shape: pl.BlockSpec(shape, lambda i: tuple(0 for _ in shape))
    return pl.pallas_call(
        _fin_body,
        grid=(10,),
        in_specs=[
            pl.BlockSpec((NC, 1000, H), lambda i: (0, i, 0)),
            pl.BlockSpec((1000, 1), lambda i: (i, 0)),
            full((NC, 1, H)),
            full((256, 256)), full((1, 256)),
            full((256, 256)), full((1, 256)),
            full((128, 256)), full((1, 128)),
            full((128, 256)), full((1, 128)),
        ],
        out_specs=[full((1, 256)), full((1, 256)), full((1, 128)), full((1, 128))],
        out_shape=[
            jax.ShapeDtypeStruct((1, 256), jnp.float32),
            jax.ShapeDtypeStruct((1, 256), jnp.float32),
            jax.ShapeDtypeStruct((1, 128), jnp.float32),
            jax.ShapeDtypeStruct((1, 128), jnp.float32),
        ],
        scratch_shapes=[pltpu.VMEM((NC, H), jnp.float32)],
    )(agg, dinv, bias2, w1, b1, w2, b2, w3, b3, w4, b4)


def kernel(x, edge_index, conv1_weight, conv1_bias, fc1_weight, fc1_bias,
           fc2_weight, fc2_bias, fc3_weight, fc3_bias, fc4_weight, fc4_bias):
    src = edge_index[0]
    dst = edge_index[1]

    degp = _deg_kernel(dst)
    degpt = degp.T
    h2, dinv = _h2_call(x, conv1_weight, degpt)
    comb = src + (dst << 14)
    agg = _scat_kernel(h2, comb.reshape(NS, NCH, CHUNK))

    bias2 = conv1_bias.reshape(NC, 1, H)
    g1, b1, g2, b2 = _fin_call(
        agg, dinv, bias2,
        fc1_weight, fc1_bias.reshape(1, -1),
        fc2_weight, fc2_bias.reshape(1, -1),
        fc3_weight, fc3_bias.reshape(1, -1),
        fc4_weight, fc4_bias.reshape(1, -1),
    )
    return (g1.reshape(-1), b1.reshape(-1), g2.reshape(-1), b2.reshape(-1))


# speedup vs baseline: 29.6936x; 1.2251x over previous
import functools

import jax
import jax.numpy as jnp
from jax import lax
from jax.experimental import pallas as pl
from jax.experimental.pallas import tpu as pltpu
from jax.experimental.pallas import tpu_sc as plsc

N = 10000
E = 320000
H = 128
NC, NS = 2, 16
NW = NC * NS

EPT = E // NW
EPS = E // NS
CHUNK = 32
NCH = EPS // CHUNK
RING = 5
NGRP = NCH // RING
RPT = 632

_MESH = plsc.VectorSubcoreMesh(
    core_axis_name="c", subcore_axis_name="s", num_cores=NC, num_subcores=NS)


@functools.partial(
    pl.kernel,
    out_type=jax.ShapeDtypeStruct((NW, N), jnp.float32),
    mesh=_MESH,
    compiler_params=pltpu.CompilerParams(needs_layout_passes=False),
    scratch_types=[
        pltpu.VMEM((EPT,), jnp.int32),
        pltpu.VMEM((N,), jnp.float32),
    ],
)
def _deg_kernel(dst_hbm, out_hbm, idx_v, acc_v):
    c = lax.axis_index("c")
    s = lax.axis_index("s")
    wid = s * NC + c

    zeros = jnp.zeros((16,), jnp.float32)

    def zbody(i, carry):
        acc_v[pl.ds(i * 16, 16)] = zeros
        return carry

    lax.fori_loop(0, N // 16, zbody, 0)

    pltpu.sync_copy(dst_hbm.at[pl.ds(wid * EPT, EPT)], idx_v)

    ones = jnp.ones((16,), jnp.float32)

    def body(i, carry):
        idx = idx_v[pl.ds(i * 16, 16)]
        plsc.addupdate_scatter(acc_v, [idx], ones)
        return carry

    lax.fori_loop(0, EPT // 16, body, 0)

    pltpu.sync_copy(acc_v, out_hbm.at[wid])


def _h2_body(x_ref, w_ref, degpt_ref, h2_ref, dinv_ref):
    deg = jnp.sum(degpt_ref[...], axis=1, keepdims=True) + 1.0
    dinv = lax.rsqrt(deg)
    h = jnp.dot(x_ref[...], w_ref[...], preferred_element_type=jnp.float32)
    h2_ref[...] = (h * dinv)[None]
    dinv_ref[...] = dinv


def _h2_call(x, w, degpt):
    return pl.pallas_call(
        _h2_body,
        grid=(10, 2),
        in_specs=[
            pl.BlockSpec((1000, 128), lambda i, c: (i, 0)),
            pl.BlockSpec((128, H), lambda i, c: (0, c)),
            pl.BlockSpec((1000, NW), lambda i, c: (i, 0)),
        ],
        out_specs=[
            pl.BlockSpec((1, 1000, H), lambda i, c: (c, i, 0)),
            pl.BlockSpec((1000, 1), lambda i, c: (i, 0)),
        ],
        out_shape=[
            jax.ShapeDtypeStruct((NC, N, H), jnp.float32),
            jax.ShapeDtypeStruct((N, 1), jnp.float32),
        ],
    )(x, w, degpt)


@functools.partial(
    pl.kernel,
    out_type=jax.ShapeDtypeStruct((NC, N, H), jnp.float32),
    mesh=_MESH,
    scratch_types=(
        [pltpu.VMEM_SHARED((N, H), jnp.float32)]
        + [pltpu.VMEM((CHUNK, H), jnp.float32) for _ in range(RING)]
        + [pltpu.VMEM((CHUNK,), jnp.int32) for _ in range(3 * RING)]
        + [pltpu.SemaphoreType.DMA for _ in range(3 * RING)]
    ),
)
def _scat_kernel(h2_hbm, combr_hbm, out_hbm, acc_s, *rest):
    rows = rest[:RING]
    cidx = rest[RING:2 * RING]
    sidx = rest[2 * RING:3 * RING]
    didx = rest[3 * RING:4 * RING]
    gsem = rest[4 * RING:5 * RING]
    ssem = rest[5 * RING:6 * RING]
    csem = rest[6 * RING:]
    c = lax.axis_index("c")
    s = lax.axis_index("s")
    off = jnp.minimum(s * RPT, N - RPT)

    pltpu.sync_copy(h2_hbm.at[c, pl.ds(off, RPT), :], acc_s.at[pl.ds(off, RPT), :])
    plsc.subcore_barrier()

    def unpack(b):
        for t in range(CHUNK // 16):
            sl = pl.ds(t * 16, 16)
            v = cidx[b][sl]
            sidx[b][sl] = jnp.bitwise_and(v, 16383)
            didx[b][sl] = lax.shift_right_logical(v, 14)

    def run(tbl):
        for b in range(RING):
            pltpu.async_copy(combr_hbm.at[s, b], cidx[b], csem[b])
        for b in range(RING):
            pltpu.make_async_copy(combr_hbm.at[s, 0], cidx[b], csem[b]).wait()
            unpack(b)
            pltpu.async_copy(tbl.at[sidx[b]], rows[b], gsem[b])
            pltpu.async_copy(combr_hbm.at[s, RING + b], cidx[b], csem[b])

        def body(k, carry):
            for b in range(RING):
                pltpu.make_async_copy(tbl.at[pl.ds(0, CHUNK)], rows[b], gsem[b]).wait()
                pltpu.async_copy(rows[b], acc_s.at[didx[b]], ssem[b], add=True)

            @pl.when(k < NGRP - 1)
            def _():
                for b in range(RING):
                    j2 = (k + 1) * RING + b
                    pltpu.make_async_copy(rows[b], acc_s.at[didx[b]], ssem[b]).wait()
                    pltpu.make_async_copy(combr_hbm.at[s, 0], cidx[b], csem[b]).wait()
                    unpack(b)
                    pltpu.async_copy(tbl.at[sidx[b]], rows[b], gsem[b])

                    @pl.when(k < NGRP - 2)
                    def _():
                        pltpu.async_copy(
                            combr_hbm.at[s, j2 + RING], cidx[b], csem[b])

            return carry

        lax.fori_loop(0, NGRP, body, 0)
        for b in range(RING):
            pltpu.make_async_copy(rows[b], acc_s.at[didx[b]], ssem[b]).wait()

    @pl.when(c == 0)
    def _():
        run(h2_hbm.at[0])

    @pl.when(c == 1)
    def _():
        run(h2_hbm.at[1])

    plsc.subcore_barrier()
    pltpu.sync_copy(acc_s.at[pl.ds(off, RPT), :], out_hbm.at[c, pl.ds(off, RPT), :])


def _fin_body(agg_ref, dinv_ref, bias_ref, w1_ref, b1_ref, w2_ref, b2_ref,
              w3_ref, b3_ref, w4_ref, b4_ref,
              g1_ref, be1_ref, g2_ref, be2_ref, s_acc):
    i = pl.program_id(0)

    @pl.when(i == 0)
    def _():
        s_acc[...] = jnp.zeros_like(s_acc)

    t = agg_ref[...] * dinv_ref[...][None] + bias_ref[...]
    t = jnp.maximum(t, 0.0)
    s_acc[...] += jnp.sum(t, axis=1)

    @pl.when(i == pl.num_programs(0) - 1)
    def _():
        s0 = s_acc[0:1, :]
        s1 = s_acc[1:2, :]

        def head(w_ref, b_ref):
            w = w_ref[...]
            dn = (((1,), (1,)), ((), ()))
            r = (lax.dot_general(s0, w[:, :H], dn, preferred_element_type=jnp.float32)
                 + lax.dot_general(s1, w[:, H:], dn, preferred_element_type=jnp.float32))
            return jnp.tanh(r + b_ref[...])

        g1_ref[...] = head(w1_ref, b1_ref)
        be1_ref[...] = head(w2_ref, b2_ref)
        g2_ref[...] = head(w3_ref, b3_ref)
        be2_ref[...] = head(w4_ref, b4_ref)


def _fin_call(agg, dinv, bias2, w1, b1, w2, b2, w3, b3, w4, b4):
    full = lambda shape: pl.BlockSpec(shape, lambda i: tuple(0 for _ in shape))
    return pl.pallas_call(
        _fin_body,
        grid=(10,),
        in_specs=[
            pl.BlockSpec((NC, 1000, H), lambda i: (0, i, 0)),
            pl.BlockSpec((1000, 1), lambda i: (i, 0)),
            full((NC, 1, H)),
            full((256, 256)), full((1, 256)),
            full((256, 256)), full((1, 256)),
            full((128, 256)), full((1, 128)),
            full((128, 256)), full((1, 128)),
        ],
        out_specs=[full((1, 256)), full((1, 256)), full((1, 128)), full((1, 128))],
        out_shape=[
            jax.ShapeDtypeStruct((1, 256), jnp.float32),
            jax.ShapeDtypeStruct((1, 256), jnp.float32),
            jax.ShapeDtypeStruct((1, 128), jnp.float32),
            jax.ShapeDtypeStruct((1, 128), jnp.float32),
        ],
        scratch_shapes=[pltpu.VMEM((NC, H), jnp.float32)],
    )(agg, dinv, bias2, w1, b1, w2, b2, w3, b3, w4, b4)


def kernel(x, edge_index, conv1_weight, conv1_bias, fc1_weight, fc1_bias,
           fc2_weight, fc2_bias, fc3_weight, fc3_bias, fc4_weight, fc4_bias):
    src = edge_index[0]
    dst = edge_index[1]

    degp = _deg_kernel(dst)
    degpt = degp.T
    h2, dinv = _h2_call(x, conv1_weight, degpt)
    comb = src + (dst << 14)
    agg = _scat_kernel(h2, comb.reshape(NS, NCH, CHUNK))

    bias2 = conv1_bias.reshape(NC, 1, H)
    g1, b1, g2, b2 = _fin_call(
        agg, dinv, bias2,
        fc1_weight, fc1_bias.reshape(1, -1),
        fc2_weight, fc2_bias.reshape(1, -1),
        fc3_weight, fc3_bias.reshape(1, -1),
        fc4_weight, fc4_bias.reshape(1, -1),
    )
    return (g1.reshape(-1), b1.reshape(-1), g2.reshape(-1), b2.reshape(-1))
